# baseline (speedup 1.0000x reference)
import jax
import jax.numpy as jnp
from jax.experimental import pallas as pl
from jax.experimental.pallas import tpu as pltpu

_K = 3
_PAD = 1
_EPS = 1e-5
_TW = 2
_BT0 = 32
_BT2 = 8


@jax.jit
def _forward(x_nchw, mask_nchw, conv_w_oihw, gamma, beta):
    B, Cin, H, W = x_nchw.shape
    Cout = conv_w_oihw.shape[0]
    Hout, Wout = H, W
    Wp = W + 2 * _PAD
    CH = Cin * H
    CW = Cout * Hout
    NWT = Wout // _TW
    M = _TW * B
    n = float(B * Hout * Wout)
    bt0 = min(_BT0, B)
    bt2 = min(_BT2, B)

    def tr_kernel(x_ref, m_ref, xs_ref, ms_ref):
        xb = x_ref[...].astype(jnp.bfloat16).reshape(bt0, CH, W)
        t1 = jnp.transpose(xb, (0, 2, 1))
        xs_ref[_PAD:_PAD + W, :, :] = jnp.transpose(t1, (1, 0, 2))
        xs_ref[0:_PAD, :, :] = jnp.zeros((_PAD, bt0, CH), jnp.bfloat16)
        xs_ref[_PAD + W:Wp, :, :] = jnp.zeros((_PAD, bt0, CH), jnp.bfloat16)
        mb = m_ref[...].astype(jnp.bfloat16).reshape(bt0, Hout, Wout)
        t2 = jnp.transpose(mb, (0, 2, 1))
        ms_ref[...] = jnp.transpose(t2, (1, 0, 2))

    x_slab, m_slab = pl.pallas_call(
        tr_kernel,
        out_shape=(jax.ShapeDtypeStruct((Wp, B, CH), jnp.bfloat16),
                   jax.ShapeDtypeStruct((Wout, B, Hout), jnp.bfloat16)),
        grid=(B // bt0,),
        in_specs=[
            pl.BlockSpec((bt0, Cin, H, W), lambda ib: (ib, 0, 0, 0)),
            pl.BlockSpec((bt0, 1, Hout, Wout), lambda ib: (ib, 0, 0, 0)),
        ],
        out_specs=(
            pl.BlockSpec((Wp, bt0, CH), lambda ib: (0, ib, 0)),
            pl.BlockSpec((Wout, bt0, Hout), lambda ib: (0, ib, 0)),
        ),
        compiler_params=pltpu.CompilerParams(
            dimension_semantics=("parallel",),
            vmem_limit_bytes=64 * 1024 * 1024),
    )(x_nchw, mask_nchw)

    wt = jnp.transpose(conv_w_oihw, (2, 3, 1, 0)).astype(jnp.float32)
    sel = jnp.stack([jnp.eye(Hout, Hout, k=kh - _PAD, dtype=jnp.float32)
                     for kh in range(_K)])
    band = jnp.einsum('xhp,xkio->kipoh', sel, wt)
    w_band = band.reshape(_K, CH, CW).astype(jnp.bfloat16)

    expand = jnp.tile(jnp.eye(Hout, dtype=jnp.bfloat16), (1, Cout))

    def conv_kernel(x_ref, w_ref, m_ref, e_ref, y_ref, st_ref):
        wt_i = pl.program_id(0)
        w0 = pl.multiple_of(wt_i * _TW, _TW)
        acc = jnp.zeros((M, CW), jnp.float32)
        for kw in range(_K):
            lhs = x_ref[pl.ds(w0 + kw, _TW), :, :].reshape(M, CH)
            acc = acc + jnp.dot(lhs, w_ref[kw],
                                preferred_element_type=jnp.float32)
        mexp = jnp.dot(m_ref[...].reshape(M, Hout), e_ref[...],
                       preferred_element_type=jnp.float32)
        ym = acc * mexp
        y_ref[...] = ym.reshape(_TW, B, CW).astype(jnp.bfloat16)
        st_ref[0, 0:1, :] = jnp.sum(ym, axis=0, keepdims=True)
        st_ref[0, 1:2, :] = jnp.sum(ym * ym, axis=0, keepdims=True)

    y, st = pl.pallas_call(
        conv_kernel,
        out_shape=(jax.ShapeDtypeStruct((Wout, B, CW), jnp.bfloat16),
                   jax.ShapeDtypeStruct((NWT, 2, CW), jnp.float32)),
        grid=(NWT,),
        in_specs=[
            pl.BlockSpec((Wp, B, CH), lambda i: (0, 0, 0)),
            pl.BlockSpec((_K, CH, CW), lambda i: (0, 0, 0)),
            pl.BlockSpec((_TW, B, Hout), lambda i: (i, 0, 0)),
            pl.BlockSpec((Hout, CW), lambda i: (0, 0)),
        ],
        out_specs=(
            pl.BlockSpec((_TW, B, CW), lambda i: (i, 0, 0)),
            pl.BlockSpec((1, 2, CW), lambda i: (i, 0, 0)),
        ),
        compiler_params=pltpu.CompilerParams(
            dimension_semantics=("parallel",),
            vmem_limit_bytes=64 * 1024 * 1024),
    )(x_slab, w_band, m_slab, expand)

    osum = jnp.kron(jnp.eye(Cout, dtype=jnp.float32),
                    jnp.ones((Hout, Hout), jnp.float32))
    rexp = jnp.repeat(jnp.eye(Cout, dtype=jnp.float32), Hout, axis=1)
    g_row = gamma.astype(jnp.float32).reshape(1, Cout)
    b_row = beta.astype(jnp.float32).reshape(1, Cout)
    eyeW = jnp.eye(Wout, dtype=jnp.float32)

    def bn_relu_kernel(y_ref, st_ref, o_ref, r_ref, g_ref, bt_ref, i_ref,
                       out_ref):
        s = jnp.sum(st_ref[...], axis=0)
        tot = jnp.dot(s, o_ref[...],
                      preferred_element_type=jnp.float32) * (1.0 / n)
        mean = tot[0:1, :]
        var = jnp.maximum(tot[1:2, :] - mean * mean, 0.0)
        gl = jnp.dot(g_ref[...], r_ref[...],
                     preferred_element_type=jnp.float32)
        bl = jnp.dot(bt_ref[...], r_ref[...],
                     preferred_element_type=jnp.float32)
        scale = gl / jnp.sqrt(var + _EPS)
        shift = bl - mean * scale
        for ib in range(bt2):
            z = jnp.maximum(
                y_ref[:, ib, :].astype(jnp.float32) * scale + shift, 0.0)
            zt = jax.lax.dot_general(
                z, i_ref[...], (((0,), (0,)), ((), ())),
                preferred_element_type=jnp.float32)
            out_ref[ib, :, :, :] = zt.reshape(Cout, Hout, Wout)

    z = pl.pallas_call(
        bn_relu_kernel,
        out_shape=jax.ShapeDtypeStruct((B, Cout, Hout, Wout), jnp.float32),
        grid=(B // bt2,),
        in_specs=[
            pl.BlockSpec((Wout, bt2, CW), lambda ib: (0, ib, 0)),
            pl.BlockSpec((NWT, 2, CW), lambda ib: (0, 0, 0)),
            pl.BlockSpec((CW, CW), lambda ib: (0, 0)),
            pl.BlockSpec((Cout, CW), lambda ib: (0, 0)),
            pl.BlockSpec((1, Cout), lambda ib: (0, 0)),
            pl.BlockSpec((1, Cout), lambda ib: (0, 0)),
            pl.BlockSpec((Wout, Wout), lambda ib: (0, 0)),
        ],
        out_specs=pl.BlockSpec((bt2, Cout, Hout, Wout), lambda ib: (ib, 0, 0, 0)),
        compiler_params=pltpu.CompilerParams(
            dimension_semantics=("parallel",),
            vmem_limit_bytes=64 * 1024 * 1024),
    )(y, st, osum, rexp, g_row, b_row, eyeW)

    return z, mask_nchw


def kernel(x_nchw, mask_nchw, conv_w_oihw, gamma, beta):
    if mask_nchw is None:
        ones = jnp.ones((x_nchw.shape[0], 1) + x_nchw.shape[2:], jnp.float32)
        out, _ = _forward(x_nchw, ones, conv_w_oihw, gamma, beta)
        return out, None
    return _forward(x_nchw, mask_nchw, conv_w_oihw, gamma, beta)


# speedup vs baseline: 1.4292x; 1.4292x over previous
import jax
import jax.numpy as jnp
import numpy as np
from jax.experimental import pallas as pl
from jax.experimental.pallas import tpu as pltpu

_K = 3
_PAD = 1
_EPS = 1e-5
_TH = 2


@jax.jit
def _forward(x_nchw, mask_nchw, conv_w_oihw, gamma, beta):
    B, Cin, H, W = x_nchw.shape
    Cout = conv_w_oihw.shape[0]
    Hout, Wout = H, W
    Hp = H + 2 * _PAD
    WC = Wout * Cin
    CW = Cout * Wout
    NHT = Hout // _TH
    M = _TH * B
    n = float(B * Hout * Wout)

    xt = jnp.transpose(x_nchw, (2, 0, 3, 1)).astype(jnp.bfloat16)
    xt = jnp.pad(xt, ((_PAD, _PAD), (0, 0), (0, 0), (0, 0)))
    x_slab = xt.reshape(Hp * B, WC)

    wt = jnp.transpose(conv_w_oihw, (2, 3, 1, 0)).astype(jnp.float32)
    sel = jnp.asarray(np.stack([np.eye(Wout, Wout, k=kw - _PAD,
                                        dtype=np.float32)
                                for kw in range(_K)]))
    band = jnp.einsum('xwp,kxio->kpiow', sel, wt)
    w_band = band.reshape(_K, WC, CW).astype(jnp.bfloat16)

    mt = jnp.transpose(mask_nchw.reshape(B, Hout, Wout), (1, 0, 2))
    m_slab = mt.reshape(Hout * B, Wout).astype(jnp.bfloat16)
    expand = jnp.asarray(np.tile(np.eye(Wout, dtype=np.float32), (1, Cout)),
                         dtype=jnp.bfloat16)

    def conv_kernel(x_ref, w_ref, m_ref, e_ref, y_ref, st_ref):
        ht = pl.program_id(0)
        r0 = pl.multiple_of(ht * M, M)
        acc = jnp.zeros((M, CW), jnp.float32)
        for kh in range(_K):
            lhs = x_ref[pl.ds(r0 + kh * B, M), :]
            acc = acc + jnp.dot(lhs, w_ref[kh],
                                preferred_element_type=jnp.float32)
        mexp = jnp.dot(m_ref[pl.ds(r0, M), :], e_ref[...],
                       preferred_element_type=jnp.float32)
        ym = acc * mexp
        y_ref[...] = ym.astype(jnp.bfloat16)
        st_ref[0, 0:1, :] = jnp.sum(ym, axis=0, keepdims=True)
        st_ref[0, 1:2, :] = jnp.sum(ym * ym, axis=0, keepdims=True)

    y, st = pl.pallas_call(
        conv_kernel,
        out_shape=(jax.ShapeDtypeStruct((Hout * B, CW), jnp.bfloat16),
                   jax.ShapeDtypeStruct((NHT, 2, CW), jnp.float32)),
        grid=(NHT,),
        in_specs=[
            pl.BlockSpec((Hp * B, WC), lambda ht: (0, 0)),
            pl.BlockSpec((_K, WC, CW), lambda ht: (0, 0, 0)),
            pl.BlockSpec((Hout * B, Wout), lambda ht: (0, 0)),
            pl.BlockSpec((Wout, CW), lambda ht: (0, 0)),
        ],
        out_specs=(
            pl.BlockSpec((M, CW), lambda ht: (ht, 0)),
            pl.BlockSpec((1, 2, CW), lambda ht: (ht, 0, 0)),
        ),
        compiler_params=pltpu.CompilerParams(
            dimension_semantics=("parallel",),
            vmem_limit_bytes=64 * 1024 * 1024),
    )(x_slab, w_band, m_slab, expand)

    osum = jnp.asarray(np.kron(np.eye(Cout, dtype=np.float32),
                               np.ones((Wout, Wout), np.float32)))
    rexp = jnp.asarray(np.repeat(np.eye(Cout, dtype=np.float32), Wout,
                                 axis=1))
    g_row = gamma.astype(jnp.float32).reshape(1, Cout)
    b_row = beta.astype(jnp.float32).reshape(1, Cout)

    R = Hout * B
    TR = min(1024, R)

    def bn_relu_kernel(y_ref, st_ref, o_ref, r_ref, g_ref, bt_ref, out_ref):
        s = jnp.sum(st_ref[...], axis=0)
        tot = jnp.dot(s, o_ref[...],
                      preferred_element_type=jnp.float32) * (1.0 / n)
        mean = tot[0:1, :]
        var = jnp.maximum(tot[1:2, :] - mean * mean, 0.0)
        gl = jnp.dot(g_ref[...], r_ref[...],
                     preferred_element_type=jnp.float32)
        bl = jnp.dot(bt_ref[...], r_ref[...],
                     preferred_element_type=jnp.float32)
        scale = gl / jnp.sqrt(var + _EPS)
        shift = bl - mean * scale
        yv = y_ref[...].astype(jnp.float32)
        out_ref[...] = jnp.maximum(yv * scale + shift, 0.0).astype(jnp.bfloat16)

    z = pl.pallas_call(
        bn_relu_kernel,
        out_shape=jax.ShapeDtypeStruct((R, CW), jnp.bfloat16),
        grid=(R // TR,),
        in_specs=[
            pl.BlockSpec((TR, CW), lambda i: (i, 0)),
            pl.BlockSpec((NHT, 2, CW), lambda i: (0, 0, 0)),
            pl.BlockSpec((CW, CW), lambda i: (0, 0)),
            pl.BlockSpec((Cout, CW), lambda i: (0, 0)),
            pl.BlockSpec((1, Cout), lambda i: (0, 0)),
            pl.BlockSpec((1, Cout), lambda i: (0, 0)),
        ],
        out_specs=pl.BlockSpec((TR, CW), lambda i: (i, 0)),
        compiler_params=pltpu.CompilerParams(
            dimension_semantics=("parallel",),
            vmem_limit_bytes=64 * 1024 * 1024),
    )(y, st, osum, rexp, g_row, b_row)

    out = z.reshape(Hout, B, Cout, Wout).transpose(1, 2, 0, 3).astype(jnp.float32)
    return out, mask_nchw


def kernel(x_nchw, mask_nchw, conv_w_oihw, gamma, beta):
    if mask_nchw is None:
        ones = jnp.ones((x_nchw.shape[0], 1) + x_nchw.shape[2:], jnp.float32)
        out, _ = _forward(x_nchw, ones, conv_w_oihw, gamma, beta)
        return out, None
    return _forward(x_nchw, mask_nchw, conv_w_oihw, gamma, beta)
